# baseline (speedup 1.0000x reference)
import functools

import jax
import jax.numpy as jnp
from jax import lax
from jax.experimental import pallas as pl
from jax.experimental.pallas import tpu as pltpu
from jax.experimental.pallas import tpu_sc as plsc

B = 16384
D0, D1 = 64, 32
D = D0 + D1
IV = 1000000
UV = 100000

NC = 2
NS = 16
NW = NC * NS
BW = B // NW
L = 16
CH = 128
NCH = BW // CH
OUTW = 384
ROWS_PER_W = B * D // OUTW // NW


def _sc_body(item_ids, user_ids, it0, ut0, it1, ut1,
             out_item, out_user,
             idx_i, idx_u, blk0, blk1, g0, g1, cat, sem):
    wid = lax.axis_index("s") * NC + lax.axis_index("c")
    base = wid * BW
    pltpu.sync_copy(item_ids.at[pl.ds(base, BW)], idx_i)
    pltpu.sync_copy(user_ids.at[pl.ds(base, BW)], idx_u)

    for idx, t0, t1, out in (
        (idx_i, it0, it1, out_item),
        (idx_u, ut0, ut1, out_user),
    ):
        def mkblk(g, _, idx=idx):
            vec = idx[pl.ds(g * L, L)]
            blk0[pl.ds(g * L, L)] = vec >> 1
            blk1[pl.ds(g * L, L)] = vec >> 2
            return ()

        lax.fori_loop(0, BW // L, mkblk, ())

        def chunk(c, _, idx=idx, t0=t0, t1=t1):
            csl = pl.ds(c * CH, CH)
            a0 = pltpu.async_copy(t0.at[blk0.at[csl]], g0, sem)
            a1 = pltpu.async_copy(t1.at[blk1.at[csl]], g1, sem)
            a0.wait()
            a1.wait()

            def agroup(gi, _):
                vec = idx[pl.ds(c * CH + gi * L, L)]
                h0 = (vec & 1) << 6
                h1 = (vec & 3) << 5
                for j in range(L):
                    s0 = pl.multiple_of(h0[j], 16)
                    s1 = pl.multiple_of(h1[j], 16)
                    gidx = gi * L + j
                    row = (c * CH + gi * L + j) // 4
                    colbase = D * (j % 4)
                    for k in range(D0 // L):
                        cat[row, pl.ds(colbase + k * L, L)] = \
                            g0[gidx, pl.ds(s0 + k * L, L)]
                    for k in range(D1 // L):
                        cat[row, pl.ds(colbase + D0 + k * L, L)] = \
                            g1[gidx, pl.ds(s1 + k * L, L)]
                return ()

            lax.fori_loop(0, CH // L, agroup, ())
            return ()

        lax.fori_loop(0, NCH, chunk, ())
        pltpu.sync_copy(cat, out.at[pl.ds(wid * ROWS_PER_W, ROWS_PER_W)])


def kernel(item_ids, user_ids, item_table_0, user_table_0, item_table_1, user_table_1):
    mesh = plsc.VectorSubcoreMesh(core_axis_name="c", subcore_axis_name="s")
    run = functools.partial(
        pl.kernel,
        out_type=(
            jax.ShapeDtypeStruct((B * D // OUTW, OUTW), jnp.float32),
            jax.ShapeDtypeStruct((B * D // OUTW, OUTW), jnp.float32),
        ),
        mesh=mesh,
        scratch_types=[
            pltpu.VMEM((BW,), jnp.int32),
            pltpu.VMEM((BW,), jnp.int32),
            pltpu.VMEM((BW,), jnp.int32),
            pltpu.VMEM((BW,), jnp.int32),
            pltpu.VMEM((CH, 128), jnp.float32),
            pltpu.VMEM((CH, 128), jnp.float32),
            pltpu.VMEM((ROWS_PER_W, OUTW), jnp.float32),
            pltpu.SemaphoreType.DMA,
        ],
    )(_sc_body)
    o_i, o_u = run(
        item_ids, user_ids,
        item_table_0.reshape(IV // 2, 128),
        user_table_0.reshape(UV // 2, 128),
        item_table_1.reshape(IV // 4, 128),
        user_table_1.reshape(UV // 4, 128),
    )
    return o_i.reshape(B, D), o_u.reshape(B, D)


# speedup vs baseline: 1.0310x; 1.0310x over previous
import functools

import jax
import jax.numpy as jnp
from jax import lax
from jax.experimental import pallas as pl
from jax.experimental.pallas import tpu as pltpu
from jax.experimental.pallas import tpu_sc as plsc

B = 16384
D0, D1 = 64, 32
D = D0 + D1

NC = 2
NS = 16
NW = NC * NS
BW = B // NW
CH = 128
NCH = BW // CH


def _sc_body(item_ids, user_ids, it0, ut0, it1, ut1,
             out_item, out_user,
             idx_i, idx_u, ri0, ri1, ru0, ru1, sem):
    wid = lax.axis_index("s") * NC + lax.axis_index("c")
    base = wid * BW
    pltpu.sync_copy(item_ids.at[pl.ds(base, BW)], idx_i)
    pltpu.sync_copy(user_ids.at[pl.ds(base, BW)], idx_u)
    gathers = []
    for c in range(NCH):
        sl = pl.ds(c * CH, CH)
        gathers.append(pltpu.async_copy(it0.at[idx_i.at[sl]], ri0.at[sl], sem))
        gathers.append(pltpu.async_copy(it1.at[idx_i.at[sl]], ri1.at[sl], sem))
        gathers.append(pltpu.async_copy(ut0.at[idx_u.at[sl]], ru0.at[sl], sem))
        gathers.append(pltpu.async_copy(ut1.at[idx_u.at[sl]], ru1.at[sl], sem))
    for g in gathers:
        g.wait()
    rows = pl.ds(base, BW)
    pltpu.sync_copy(ri0, out_item.at[rows, pl.ds(0, D0)])
    pltpu.sync_copy(ri1, out_item.at[rows, pl.ds(D0, D1)])
    pltpu.sync_copy(ru0, out_user.at[rows, pl.ds(0, D0)])
    pltpu.sync_copy(ru1, out_user.at[rows, pl.ds(D0, D1)])


def kernel(item_ids, user_ids, item_table_0, user_table_0, item_table_1, user_table_1):
    mesh = plsc.VectorSubcoreMesh(core_axis_name="c", subcore_axis_name="s")
    run = functools.partial(
        pl.kernel,
        out_type=(
            jax.ShapeDtypeStruct((B, D), jnp.float32),
            jax.ShapeDtypeStruct((B, D), jnp.float32),
        ),
        mesh=mesh,
        scratch_types=[
            pltpu.VMEM((BW,), jnp.int32),
            pltpu.VMEM((BW,), jnp.int32),
            pltpu.VMEM((BW, D0), jnp.float32),
            pltpu.VMEM((BW, D1), jnp.float32),
            pltpu.VMEM((BW, D0), jnp.float32),
            pltpu.VMEM((BW, D1), jnp.float32),
            pltpu.SemaphoreType.DMA,
        ],
        compiler_params=pltpu.CompilerParams(use_tc_tiling_on_sc=False),
    )(_sc_body)
    return run(item_ids, user_ids, item_table_0, user_table_0,
               item_table_1, user_table_1)


# speedup vs baseline: 1.2946x; 1.2557x over previous
import functools

import jax
import jax.numpy as jnp
from jax import lax
from jax.experimental import pallas as pl
from jax.experimental.pallas import tpu as pltpu
from jax.experimental.pallas import tpu_sc as plsc

B = 16384
D0, D1 = 64, 32
D = D0 + D1

NC = 2
NS = 16
NW = NC * NS
BW = B // NW
L = 16
CH = 32
NCH = BW // CH
OUTW = 384
ROWS_PER_W = B * D // OUTW // NW


def _sc_body(item_ids, user_ids, it0, ut0, it1, ut1,
             out_item, out_user,
             idx_i, idx_u, g0, g1, cat, sem):
    wid = lax.axis_index("s") * NC + lax.axis_index("c")
    base = wid * BW
    pltpu.sync_copy(item_ids.at[pl.ds(base, BW)], idx_i)
    pltpu.sync_copy(user_ids.at[pl.ds(base, BW)], idx_u)

    for idx, t0, t1, out in (
        (idx_i, it0, it1, out_item),
        (idx_u, ut0, ut1, out_user),
    ):
        def chunk(c, _, idx=idx, t0=t0, t1=t1):
            def issue(gi, _):
                vec = idx[pl.ds(c * CH + gi * L, L)]
                blk = (vec >> 3) << 3
                for j in range(L):
                    b = pl.multiple_of(blk[j], 8)
                    i = gi * L + j
                    pltpu.async_copy(t0.at[pl.ds(b, 8)],
                                     g0.at[pl.ds(i * 8, 8)], sem)
                    pltpu.async_copy(t1.at[pl.ds(b, 8)],
                                     g1.at[pl.ds(i * 8, 8)], sem)
                return ()

            lax.fori_loop(0, CH // L, issue, ())
            pltpu.make_async_copy(t0.at[pl.ds(0, CH * 8)], g0, sem).wait()
            pltpu.make_async_copy(t1.at[pl.ds(0, CH * 8)], g1, sem).wait()

            def agroup(gi, _):
                svec = idx[pl.ds(c * CH + gi * L, L)] & 7
                for j in range(L):
                    s = svec[j]
                    r0 = (gi * L + j) * 8 + s
                    row = (c * CH + gi * L + j) // 4
                    colbase = D * (j % 4)
                    for k in range(D0 // L):
                        cat[row, pl.ds(colbase + k * L, L)] = \
                            g0[r0, pl.ds(k * L, L)]
                    for k in range(D1 // L):
                        cat[row, pl.ds(colbase + D0 + k * L, L)] = \
                            g1[r0, pl.ds(k * L, L)]
                return ()

            lax.fori_loop(0, CH // L, agroup, ())
            return ()

        lax.fori_loop(0, NCH, chunk, ())
        pltpu.sync_copy(cat, out.at[pl.ds(wid * ROWS_PER_W, ROWS_PER_W)])


def kernel(item_ids, user_ids, item_table_0, user_table_0, item_table_1, user_table_1):
    mesh = plsc.VectorSubcoreMesh(core_axis_name="c", subcore_axis_name="s")
    run = functools.partial(
        pl.kernel,
        out_type=(
            jax.ShapeDtypeStruct((B * D // OUTW, OUTW), jnp.float32),
            jax.ShapeDtypeStruct((B * D // OUTW, OUTW), jnp.float32),
        ),
        mesh=mesh,
        scratch_types=[
            pltpu.VMEM((BW,), jnp.int32),
            pltpu.VMEM((BW,), jnp.int32),
            pltpu.VMEM((CH * 8, D0), jnp.float32),
            pltpu.VMEM((CH * 8, D1), jnp.float32),
            pltpu.VMEM((ROWS_PER_W, OUTW), jnp.float32),
            pltpu.SemaphoreType.DMA,
        ],
    )(_sc_body)
    o_i, o_u = run(item_ids, user_ids, item_table_0, user_table_0,
                   item_table_1, user_table_1)
    return o_i.reshape(B, D), o_u.reshape(B, D)


# speedup vs baseline: 1.3534x; 1.0454x over previous
import functools

import jax
import jax.numpy as jnp
from jax import lax
from jax.experimental import pallas as pl
from jax.experimental.pallas import tpu as pltpu
from jax.experimental.pallas import tpu_sc as plsc

B = 16384
D0, D1 = 64, 32
D = D0 + D1

NC = 2
NS = 16
NW = NC * NS
BW = B // NW
L = 16
CH = 32
NCH = BW // CH
OUTW = 384
ROWS_PER_W = B * D // OUTW // NW


def _sc_pair_body(ids, t0, t1, out, idx, g0, g1, cat, sem):
    wid = lax.axis_index("s") * NC + lax.axis_index("c")
    base = wid * BW
    pltpu.sync_copy(ids.at[pl.ds(base, BW)], idx)

    def chunk(c, _):
        def issue(gi, _):
            vec = idx[pl.ds(c * CH + gi * L, L)]
            blk = (vec >> 3) << 3
            for j in range(L):
                b = pl.multiple_of(blk[j], 8)
                i = gi * L + j
                pltpu.async_copy(t0.at[pl.ds(b, 8)],
                                 g0.at[pl.ds(i * 8, 8)], sem)
                pltpu.async_copy(t1.at[pl.ds(b, 8)],
                                 g1.at[pl.ds(i * 8, 8)], sem)
            return ()

        lax.fori_loop(0, CH // L, issue, ())
        pltpu.make_async_copy(t0.at[pl.ds(0, CH * 8)], g0, sem).wait()
        pltpu.make_async_copy(t1.at[pl.ds(0, CH * 8)], g1, sem).wait()

        def agroup(gi, _):
            svec = idx[pl.ds(c * CH + gi * L, L)] & 7
            for j in range(L):
                s = svec[j]
                r0 = (gi * L + j) * 8 + s
                row = (c * CH + gi * L + j) // 4
                colbase = D * (j % 4)
                for k in range(D0 // L):
                    cat[row, pl.ds(colbase + k * L, L)] = \
                        g0[r0, pl.ds(k * L, L)]
                for k in range(D1 // L):
                    cat[row, pl.ds(colbase + D0 + k * L, L)] = \
                        g1[r0, pl.ds(k * L, L)]
            return ()

        lax.fori_loop(0, CH // L, agroup, ())
        return ()

    lax.fori_loop(0, NCH, chunk, ())
    pltpu.sync_copy(cat, out.at[pl.ds(wid * ROWS_PER_W, ROWS_PER_W)])


def _make_pair_kernel():
    mesh = plsc.VectorSubcoreMesh(core_axis_name="c", subcore_axis_name="s")
    return functools.partial(
        pl.kernel,
        out_type=jax.ShapeDtypeStruct((B * D // OUTW, OUTW), jnp.float32),
        mesh=mesh,
        scratch_types=[
            pltpu.VMEM((BW,), jnp.int32),
            pltpu.VMEM((CH * 8, D0), jnp.float32),
            pltpu.VMEM((CH * 8, D1), jnp.float32),
            pltpu.VMEM((ROWS_PER_W, OUTW), jnp.float32),
            pltpu.SemaphoreType.DMA,
        ],
    )(_sc_pair_body)


def kernel(item_ids, user_ids, item_table_0, user_table_0, item_table_1, user_table_1):
    run = _make_pair_kernel()
    o_u = run(user_ids, user_table_0, user_table_1)
    o_i = run(item_ids, item_table_0, item_table_1)
    return o_i.reshape(B, D), o_u.reshape(B, D)
